# baseline (speedup 1.0000x reference)
import functools
import jax
import jax.numpy as jnp
from jax import lax
from jax.experimental import pallas as pl
from jax.experimental.pallas import tpu as pltpu
from jax.experimental.pallas import tpu_sc as plsc

B = 4096
L = 200
D = 64
V = 200001
CBLK = 2048
VPAD = 200704
CH = 2
SLABL = 8
NCHUNK = L // CH
CPS = SLABL // CH
NSLAB = L // SLABL


def _norm_kernel(wt_ref, o_ref):
    w = wt_ref[...]
    s = jnp.sum(w * w, axis=0, keepdims=True)
    n = jnp.maximum(jnp.sqrt(s), 1e-12)
    wn = w / n
    eye = jnp.eye(D, dtype=jnp.float32)
    t = lax.dot_general(wn, eye, (((0,), (0,)), ((), ())),
                        preferred_element_type=jnp.float32)
    o_ref[:, pl.ds(0, D)] = t


def _normalize_table(wt):
    return pl.pallas_call(
        _norm_kernel,
        grid=(VPAD // CBLK,),
        in_specs=[pl.BlockSpec((D, CBLK), lambda i: (0, i))],
        out_specs=pl.BlockSpec((CBLK, 128), lambda i: (i, 0)),
        out_shape=jax.ShapeDtypeStruct((VPAD, 128), jnp.float32),
    )(wt)


def _sc_kernel(rid_hbm, e_hbm, tnorm_hbm, out_hbm,
               idx_v, w_v, e_v, acc_v, sems, slab_sem):
    info = plsc.get_sparse_core_info()
    nc = info.num_cores
    wid = lax.axis_index("s") * nc + lax.axis_index("c")

    lanes = jnp.arange(16, dtype=jnp.int32)

    def slab_start(si):
        return pltpu.async_copy(rid_hbm.at[si, wid], idx_v.at[si % 2],
                                slab_sem)

    def issue(k, slot):
        si = k // CPS
        for u in range(CH):
            pltpu.async_copy(
                tnorm_hbm.at[idx_v.at[si % 2, (k % CPS) * CH + u]],
                w_v.at[slot].at[pl.ds(u * 128, 128)], sems.at[slot])
        pltpu.async_copy(e_hbm.at[pl.ds(k * CH, CH), :, wid],
                         e_v.at[slot], sems.at[slot])

    def drain(slot):
        for u in range(CH):
            pltpu.make_async_copy(tnorm_hbm.at[pl.ds(0, 128)],
                                  w_v.at[slot].at[pl.ds(u * 128, 128)],
                                  sems.at[slot]).wait()
        pltpu.make_async_copy(e_hbm.at[pl.ds(0, CH), :, 0],
                              e_v.at[slot], sems.at[slot]).wait()

    def slab_wait():
        pltpu.make_async_copy(rid_hbm.at[0, 0], idx_v.at[0],
                              slab_sem).wait()

    z = jnp.zeros((16,), jnp.float32)

    def zero_body(dt, _):
        for ds_ in range(8):
            for g in range(8):
                acc_v[dt, ds_, pl.ds(g * 16, 16)] = z
        return 0

    lax.fori_loop(0, 8, zero_body, 0)

    def compute(slot, k):
        wb = w_v.at[slot]
        eb = e_v.at[slot]
        for g in range(8):
            gs = g * 16
            rows = [jnp.full((16,), u * 128 + gs, jnp.int32) + lanes
                    for u in range(CH)]

            def p1(u):
                def body(d, c):
                    dd = jnp.full((16,), d, jnp.int32)
                    ev = eb[u, d // 8, d % 8, pl.ds(gs, 16)]
                    wv = plsc.load_gather(wb, [rows[u], dd])
                    return c + ev * wv
                return lax.fori_loop(0, D, body, z)

            c0 = p1(0)
            c1 = p1(1)

            def p2(d, _):
                dd = jnp.full((16,), d, jnp.int32)
                dt = d // 8
                ds_ = d % 8
                a = acc_v[dt, ds_, pl.ds(gs, 16)]
                ev0 = eb[0, dt, ds_, pl.ds(gs, 16)]
                wv0 = plsc.load_gather(wb, [rows[0], dd])
                ev1 = eb[1, dt, ds_, pl.ds(gs, 16)]
                wv1 = plsc.load_gather(wb, [rows[1], dd])
                acc_v[dt, ds_, pl.ds(gs, 16)] = (
                    a + (ev0 - c0 * wv0) + (ev1 - c1 * wv1))
                return 0

            lax.fori_loop(0, D, p2, 0)

    slab_start(0).wait()
    slab_start(1)
    issue(0, 0)

    def pair_body(p, _):
        for s2 in range(2):
            k = p * 2 + s2
            drain(s2)

            @pl.when(jnp.logical_and(k % CPS == CPS - 1, k + 1 < NCHUNK))
            def _():
                slab_wait()

            @pl.when(k + 1 < NCHUNK)
            def _():
                issue(k + 1, (s2 + 1) % 2)

            @pl.when(jnp.logical_and(k % CPS == CPS - 1,
                                     k // CPS + 2 < NSLAB))
            def _():
                slab_start(k // CPS + 2)

            compute(s2, k)
        return 0

    lax.fori_loop(0, NCHUNK // 2, pair_body, 0)

    inv = jnp.float32(1.0 / L)

    def scale_body(dt, _):
        for ds_ in range(8):
            for g in range(8):
                acc_v[dt, ds_, pl.ds(g * 16, 16)] = (
                    acc_v[dt, ds_, pl.ds(g * 16, 16)] * inv)
        return 0

    lax.fori_loop(0, 8, scale_body, 0)
    pltpu.sync_copy(acc_v, out_hbm.at[:, wid])


@jax.jit
def _run(rid4, e5, table_t):
    tnorm = _normalize_table(table_t)
    mesh = plsc.VectorSubcoreMesh(core_axis_name="c", subcore_axis_name="s")
    kfn = functools.partial(
        pl.kernel,
        mesh=mesh,
        compiler_params=pltpu.CompilerParams(use_tc_tiling_on_sc=False,
                                             needs_layout_passes=False),
        out_type=jax.ShapeDtypeStruct((8, 32, 8, 128), jnp.float32),
        scratch_types=[
            pltpu.VMEM((2, SLABL, 128), jnp.int32),
            pltpu.VMEM((2, CH * 128, 128), jnp.float32),
            pltpu.VMEM((2, CH, 8, 8, 128), jnp.float32),
            pltpu.VMEM((8, 8, 128), jnp.float32),
            pltpu.SemaphoreType.DMA((2,)),
            pltpu.SemaphoreType.DMA,
        ],
    )(_sc_kernel)
    return kfn(rid4, e5, tnorm)


def kernel(batch_nei_rid, batch_nei_e_emb, w_r_table):
    rid4 = (batch_nei_rid.T.reshape(25, 8, 32, 128)
            .transpose(0, 2, 1, 3))
    e5 = (jnp.transpose(batch_nei_e_emb, (1, 2, 0))
          .reshape(L, 8, 8, 32, 128)
          .transpose(0, 1, 3, 2, 4))
    table_t = w_r_table.T
    out4 = _run(rid4, e5, table_t)
    return out4.transpose(0, 2, 1, 3).reshape(D, B).T


# speedup vs baseline: 1.2035x; 1.2035x over previous
import functools
import jax
import jax.numpy as jnp
from jax import lax
from jax.experimental import pallas as pl
from jax.experimental.pallas import tpu as pltpu
from jax.experimental.pallas import tpu_sc as plsc

B = 4096
L = 200
D = 64
V = 200001
CBLK = 2048
VPAD = 200704
CH = 2
SLABL = 8
NCHUNK = L // CH
CPS = SLABL // CH
NSLAB = L // SLABL


def _norm_kernel(wt_ref, o_ref):
    w = wt_ref[...]
    s = jnp.sum(w * w, axis=0, keepdims=True)
    n = jnp.maximum(jnp.sqrt(s), 1e-12)
    wn = w / n
    eye = jnp.eye(D, dtype=jnp.float32)
    t = lax.dot_general(wn, eye, (((0,), (0,)), ((), ())),
                        preferred_element_type=jnp.float32)
    o_ref[:, pl.ds(0, D)] = t


def _normalize_table(wt):
    return pl.pallas_call(
        _norm_kernel,
        grid=(VPAD // CBLK,),
        in_specs=[pl.BlockSpec((D, CBLK), lambda i: (0, i))],
        out_specs=pl.BlockSpec((CBLK, 128), lambda i: (i, 0)),
        out_shape=jax.ShapeDtypeStruct((VPAD, 128), jnp.float32),
    )(wt)


def _sc_kernel(rid_hbm, e_hbm, tnorm_hbm, out_hbm,
               idx_v, w_v, e_v, acc_v, sems, slab_sem):
    info = plsc.get_sparse_core_info()
    nc = info.num_cores
    wid = lax.axis_index("s") * nc + lax.axis_index("c")

    lanes = jnp.arange(16, dtype=jnp.int32)

    def slab_start(si):
        return pltpu.async_copy(rid_hbm.at[si, wid], idx_v.at[si % 2],
                                slab_sem)

    def issue(k, slot):
        si = k // CPS
        for u in range(CH):
            pltpu.async_copy(
                tnorm_hbm.at[idx_v.at[si % 2, (k % CPS) * CH + u]],
                w_v.at[slot].at[pl.ds(u * 128, 128)], sems.at[slot])
        pltpu.async_copy(e_hbm.at[pl.ds(k * CH, CH), :, wid],
                         e_v.at[slot], sems.at[slot])

    def drain(slot):
        for u in range(CH):
            pltpu.make_async_copy(tnorm_hbm.at[pl.ds(0, 128)],
                                  w_v.at[slot].at[pl.ds(u * 128, 128)],
                                  sems.at[slot]).wait()
        pltpu.make_async_copy(e_hbm.at[pl.ds(0, CH), :, 0],
                              e_v.at[slot], sems.at[slot]).wait()

    def slab_wait():
        pltpu.make_async_copy(rid_hbm.at[0, 0], idx_v.at[0],
                              slab_sem).wait()

    z = jnp.zeros((16,), jnp.float32)

    def zero_body(dt, _):
        for ds_ in range(8):
            for g in range(8):
                acc_v[dt, ds_, pl.ds(g * 16, 16)] = z
        return 0

    lax.fori_loop(0, 8, zero_body, 0)

    def compute(slot, k):
        wb = w_v.at[slot]
        eb = e_v.at[slot]
        for g in range(8):
            gs = g * 16
            rows = [jnp.full((16,), u * 128 + gs, jnp.int32) + lanes
                    for u in range(CH)]

            def p1(u):
                def body(dt, carry):
                    ca, cb = carry
                    db = jnp.full((16,), dt * 8, jnp.int32)
                    for ds_ in range(8):
                        ev = eb[u, dt, ds_, pl.ds(gs, 16)]
                        wv = plsc.load_gather(wb, [rows[u], db + ds_])
                        if ds_ % 2 == 0:
                            ca = ca + ev * wv
                        else:
                            cb = cb + ev * wv
                    return ca, cb
                ca, cb = lax.fori_loop(0, 8, body, (z, z))
                return ca + cb

            c0 = p1(0)
            c1 = p1(1)

            def p2(dt, _):
                db = jnp.full((16,), dt * 8, jnp.int32)
                for ds_ in range(8):
                    a = acc_v[dt, ds_, pl.ds(gs, 16)]
                    ev0 = eb[0, dt, ds_, pl.ds(gs, 16)]
                    wv0 = plsc.load_gather(wb, [rows[0], db + ds_])
                    ev1 = eb[1, dt, ds_, pl.ds(gs, 16)]
                    wv1 = plsc.load_gather(wb, [rows[1], db + ds_])
                    acc_v[dt, ds_, pl.ds(gs, 16)] = (
                        a + (ev0 - c0 * wv0) + (ev1 - c1 * wv1))
                return 0

            lax.fori_loop(0, 8, p2, 0)

    slab_start(0).wait()
    slab_start(1)
    issue(0, 0)

    def pair_body(p, _):
        for s2 in range(2):
            k = p * 2 + s2
            drain(s2)

            @pl.when(jnp.logical_and(k % CPS == CPS - 1, k + 1 < NCHUNK))
            def _():
                slab_wait()

            @pl.when(k + 1 < NCHUNK)
            def _():
                issue(k + 1, (s2 + 1) % 2)

            @pl.when(jnp.logical_and(k % CPS == CPS - 1,
                                     k // CPS + 2 < NSLAB))
            def _():
                slab_start(k // CPS + 2)

            compute(s2, k)
        return 0

    lax.fori_loop(0, NCHUNK // 2, pair_body, 0)

    inv = jnp.float32(1.0 / L)

    def scale_body(dt, _):
        for ds_ in range(8):
            for g in range(8):
                acc_v[dt, ds_, pl.ds(g * 16, 16)] = (
                    acc_v[dt, ds_, pl.ds(g * 16, 16)] * inv)
        return 0

    lax.fori_loop(0, 8, scale_body, 0)
    pltpu.sync_copy(acc_v, out_hbm.at[:, wid])


@jax.jit
def _run(rid4, e5, table_t):
    tnorm = _normalize_table(table_t)
    mesh = plsc.VectorSubcoreMesh(core_axis_name="c", subcore_axis_name="s")
    kfn = functools.partial(
        pl.kernel,
        mesh=mesh,
        compiler_params=pltpu.CompilerParams(use_tc_tiling_on_sc=False,
                                             needs_layout_passes=False),
        out_type=jax.ShapeDtypeStruct((8, 32, 8, 128), jnp.float32),
        scratch_types=[
            pltpu.VMEM((2, SLABL, 128), jnp.int32),
            pltpu.VMEM((2, CH * 128, 128), jnp.float32),
            pltpu.VMEM((2, CH, 8, 8, 128), jnp.float32),
            pltpu.VMEM((8, 8, 128), jnp.float32),
            pltpu.SemaphoreType.DMA((2,)),
            pltpu.SemaphoreType.DMA,
        ],
    )(_sc_kernel)
    return kfn(rid4, e5, tnorm)


def kernel(batch_nei_rid, batch_nei_e_emb, w_r_table):
    rid4 = (batch_nei_rid.T.reshape(25, 8, 32, 128)
            .transpose(0, 2, 1, 3))
    e5 = (jnp.transpose(batch_nei_e_emb, (1, 2, 0))
          .reshape(L, 8, 8, 32, 128)
          .transpose(0, 1, 3, 2, 4))
    table_t = w_r_table.T
    out4 = _run(rid4, e5, table_t)
    return out4.transpose(0, 2, 1, 3).reshape(D, B).T


# speedup vs baseline: 3.1799x; 2.6421x over previous
import functools
import jax
import jax.numpy as jnp
from jax import lax
from jax.experimental import pallas as pl
from jax.experimental.pallas import tpu as pltpu
from jax.experimental.pallas import tpu_sc as plsc

B = 4096
L = 200
D = 64
CHUNKS = ((0, 104), (104, 96))
NBUF = 2


def _sc_kernel(rid_hbm, e_hbm, table_hbm, out_hbm,
               idx_all, w_v, e_v, o_v, sems):
    info = plsc.get_sparse_core_info()
    nc = info.num_cores
    wid = lax.axis_index("s") * nc + lax.axis_index("c")
    b_per_w = B // (nc * info.num_subcores)
    base = wid * b_per_w

    pltpu.sync_copy(rid_hbm.at[pl.ds(base * L, b_per_w * L)], idx_all)

    def issue(bi, slot):
        for off, c in CHUNKS:
            pltpu.async_copy(
                table_hbm.at[idx_all.at[pl.ds(bi * L + off, c)]],
                w_v.at[slot].at[pl.ds(off, c)], sems.at[slot])
        pltpu.async_copy(e_hbm.at[base + bi], e_v.at[slot], sems.at[slot])

    def drain(slot):
        for off, c in CHUNKS:
            pltpu.make_async_copy(e_hbm.at[0].at[pl.ds(0, c)],
                                  w_v.at[slot].at[pl.ds(off, c)],
                                  sems.at[slot]).wait()
        pltpu.make_async_copy(e_hbm.at[0], e_v.at[slot],
                              sems.at[slot]).wait()

    rot = [(jnp.arange(16, dtype=jnp.int32) + sh) & 15 for sh in (8, 4, 2, 1)]
    dnums = lax.GatherDimensionNumbers(
        offset_dims=(), collapsed_slice_dims=(0,), start_index_map=(0,))

    def _allsum(v):
        for idx in rot:
            p = lax.gather(v, idx[:, None], dnums, (1,),
                           mode=lax.GatherScatterMode.PROMISE_IN_BOUNDS)
            v = v + p
        return v

    def compute(slot, b):
        wb = w_v.at[slot]
        eb = e_v.at[slot]

        def l_body(l2, carry):
            a0, a1, a2, a3 = carry
            for u in range(2):
                l = l2 * 2 + u
                w0 = wb[l, pl.ds(0, 16)]
                w1 = wb[l, pl.ds(16, 16)]
                w2 = wb[l, pl.ds(32, 16)]
                w3 = wb[l, pl.ds(48, 16)]
                e0 = eb[l, pl.ds(0, 16)]
                e1 = eb[l, pl.ds(16, 16)]
                e2 = eb[l, pl.ds(32, 16)]
                e3 = eb[l, pl.ds(48, 16)]
                s = _allsum(w0 * w0 + w1 * w1 + w2 * w2 + w3 * w3)
                d = _allsum(e0 * w0 + e1 * w1 + e2 * w2 + e3 * w3)
                coef = d / jnp.maximum(s, 1e-24)
                a0 = a0 + (e0 - coef * w0)
                a1 = a1 + (e1 - coef * w1)
                a2 = a2 + (e2 - coef * w2)
                a3 = a3 + (e3 - coef * w3)
            return (a0, a1, a2, a3)

        z = jnp.zeros((16,), jnp.float32)
        a0, a1, a2, a3 = lax.fori_loop(0, L // 2, l_body, (z, z, z, z))
        inv = jnp.float32(1.0 / L)
        o_v[pl.ds(0, 16)] = a0 * inv
        o_v[pl.ds(16, 16)] = a1 * inv
        o_v[pl.ds(32, 16)] = a2 * inv
        o_v[pl.ds(48, 16)] = a3 * inv
        pltpu.sync_copy(o_v, out_hbm.at[pl.ds(b * D, D)])

    issue(0, 0)

    def pair_body(p, _):
        for s2 in range(NBUF):
            bi = p * NBUF + s2
            drain(s2)

            @pl.when(bi + 1 < b_per_w)
            def _():
                issue(bi + 1, (s2 + 1) % NBUF)

            compute(s2, base + bi)
        return 0

    lax.fori_loop(0, b_per_w // NBUF, pair_body, 0)


@jax.jit
def _run(rid_flat, batch_nei_e_emb, w_r_table):
    mesh = plsc.VectorSubcoreMesh(core_axis_name="c", subcore_axis_name="s")
    kfn = functools.partial(
        pl.kernel,
        mesh=mesh,
        compiler_params=pltpu.CompilerParams(use_tc_tiling_on_sc=False),
        out_type=jax.ShapeDtypeStruct((B * D,), jnp.float32),
        scratch_types=[
            pltpu.VMEM((B // 32 * L,), jnp.int32),
            pltpu.VMEM((NBUF, L, D), jnp.float32),
            pltpu.VMEM((NBUF, L, D), jnp.float32),
            pltpu.VMEM((D,), jnp.float32),
            pltpu.SemaphoreType.DMA((NBUF,)),
        ],
    )(_sc_kernel)
    return kfn(rid_flat, batch_nei_e_emb, w_r_table).reshape(B, D)


def kernel(batch_nei_rid, batch_nei_e_emb, w_r_table):
    return _run(batch_nei_rid.reshape(-1), batch_nei_e_emb, w_r_table)
